# Initial kernel scaffold; baseline (speedup 1.0000x reference)
import jax
import jax.numpy as jnp
from jax.experimental import pallas as pl


def kernel(features, conv1_W, conv1_b, conv2_W, conv2_b, conv3_W, conv3_b, dense_W, dense_b, out_W, out_b):
    raise NotImplementedError("write your pallas kernel here")


# speedup vs baseline: 925.0746x; 925.0746x over previous
import jax
import jax.numpy as jnp
from jax.experimental import pallas as pl

_ALPHA = 0.3


def _split(a):
    hi = a.astype(jnp.bfloat16)
    lo = (a - hi.astype(jnp.float32)).astype(jnp.bfloat16)
    return hi, lo


def _dot_bf16(a, b):
    return jax.lax.dot_general(a, b, (((1,), (0,)), ((), ())),
                               preferred_element_type=jnp.float32)


def _dot3(a_split, b_split):
    a_hi, a_lo = a_split
    b_hi, b_lo = b_split
    return (_dot_bf16(a_hi, b_hi) + _dot_bf16(a_hi, b_lo)
            + _dot_bf16(a_lo, b_hi))


def _fwd_kernel(feat_ref, w1_ref, b1_ref, w2_ref, b2_ref, w3_ref, b3_ref,
                dw_ref, db_ref, ow_ref, ob_ref, out_ref):
    f32 = jnp.float32
    hi_prec = jax.lax.Precision.HIGHEST
    nb, n, _ = feat_ref.shape

    ri = jax.lax.broadcasted_iota(jnp.int32, (n, n), 0)
    ci = jax.lax.broadcasted_iota(jnp.int32, (n, n), 1)
    lower = (ri > ci).astype(f32)
    eye = (ri == ci).astype(f32)

    w1s = _split(w1_ref[...])
    w2s = _split(w2_ref[...])
    w3s = _split(w3_ref[...])
    b1 = b1_ref[...]
    b2 = b2_ref[...]
    b3 = b3_ref[...]
    dws = _split(dw_ref[...])
    db = db_ref[...]
    ow = ow_ref[...]
    ob = ob_ref[...]

    def layer(a_split, x, w_split, bvec):
        agg = _dot3(a_split, _split(x))
        h = _ALPHA * x + (1.0 - _ALPHA) * agg
        z = _dot3(_split(h), w_split) + bvec
        return jnp.tanh(z)

    outs = []
    for b in range(nb):
        x0 = feat_ref[b]
        r2 = jnp.sum(x0 * x0, axis=1, keepdims=True)
        g = jax.lax.dot_general(x0, x0, (((1,), (1,)), ((), ())),
                                precision=hi_prec, preferred_element_type=f32)
        d2 = r2 + jnp.transpose(r2) - 2.0 * g
        dist = jnp.sqrt(jnp.maximum(d2, 1e-12))
        dist_l = dist * lower
        deg = jnp.sum(dist_l, axis=1, keepdims=True) + 1.0
        dinv = jax.lax.rsqrt(deg)
        a_mat = (dinv * jnp.transpose(dinv)) * (dist_l + eye)
        a_split = _split(a_mat)

        x1 = layer(a_split, x0, w1s, b1)
        x2 = layer(a_split, x1, w2s, b2)
        x3 = layer(a_split, x2, w3s, b3)

        pooled = jnp.mean(x3, axis=0, keepdims=True)
        h = jnp.tanh(_dot3(_split(pooled), dws) + db)
        out = jnp.dot(h, ow, precision=hi_prec,
                      preferred_element_type=f32) + ob
        outs.append(out)

    out_ref[...] = jnp.concatenate(outs, axis=0)


def kernel(features, conv1_W, conv1_b, conv2_W, conv2_b, conv3_W, conv3_b,
           dense_W, dense_b, out_W, out_b):
    nb = features.shape[0]
    return pl.pallas_call(
        _fwd_kernel,
        out_shape=jax.ShapeDtypeStruct((nb, 2), jnp.float32),
    )(features,
      conv1_W, conv1_b.reshape(1, -1),
      conv2_W, conv2_b.reshape(1, -1),
      conv3_W, conv3_b.reshape(1, -1),
      dense_W, dense_b.reshape(1, -1),
      out_W, out_b.reshape(1, -1))


# speedup vs baseline: 1084.0744x; 1.1719x over previous
import jax
import jax.numpy as jnp
from jax.experimental import pallas as pl

_ALPHA = 0.3


def _split(a):
    hi = a.astype(jnp.bfloat16)
    lo = (a - hi.astype(jnp.float32)).astype(jnp.bfloat16)
    return hi, lo


def _dot_bf16(a, b):
    return jax.lax.dot_general(a, b, (((1,), (0,)), ((), ())),
                               preferred_element_type=jnp.float32)


def _dot3(a_split, b_split):
    a_hi, a_lo = a_split
    b_hi, b_lo = b_split
    return (_dot_bf16(a_hi, b_hi) + _dot_bf16(a_hi, b_lo)
            + _dot_bf16(a_lo, b_hi))


def _fwd_kernel(feat_ref, w1_ref, b1_ref, w2_ref, b2_ref, w3_ref, b3_ref,
                dw_ref, db_ref, ow_ref, ob_ref, out_ref):
    f32 = jnp.float32
    hi_prec = jax.lax.Precision.HIGHEST
    nb, n, _ = feat_ref.shape

    ri = jax.lax.broadcasted_iota(jnp.int32, (n, n), 0)
    ci = jax.lax.broadcasted_iota(jnp.int32, (n, n), 1)
    lower = (ri > ci).astype(f32)
    eye = (ri == ci).astype(f32)

    w1 = w1_ref[...]
    w2 = w2_ref[...]
    w3 = w3_ref[...]
    b1 = b1_ref[...]
    b2 = b2_ref[...]
    b3 = b3_ref[...]
    dw = dw_ref[...]
    db = db_ref[...]
    ow = ow_ref[...]
    ob = ob_ref[...]

    def layer(a_split, x, w, bvec):
        agg = _dot3(a_split, _split(x))
        h = _ALPHA * x + (1.0 - _ALPHA) * agg
        z = jnp.dot(h, w, preferred_element_type=f32) + bvec
        return jnp.tanh(z)

    outs = []
    for b in range(nb):
        x0 = feat_ref[b]
        r2 = jnp.sum(x0 * x0, axis=1, keepdims=True)
        g = jax.lax.dot_general(x0, x0, (((1,), (1,)), ((), ())),
                                precision=hi_prec, preferred_element_type=f32)
        d2 = r2 + jnp.transpose(r2) - 2.0 * g
        dist = jnp.sqrt(jnp.maximum(d2, 1e-12))
        dist_l = dist * lower
        deg = jnp.sum(dist_l, axis=1, keepdims=True) + 1.0
        dinv = jax.lax.rsqrt(deg)
        a_mat = (dinv * jnp.transpose(dinv)) * (dist_l + eye)
        a_split = _split(a_mat)

        x1 = layer(a_split, x0, w1, b1)
        x2 = layer(a_split, x1, w2, b2)
        x3 = layer(a_split, x2, w3, b3)

        pooled = jnp.mean(x3, axis=0, keepdims=True)
        h = jnp.tanh(jnp.dot(pooled, dw, preferred_element_type=f32) + db)
        out = jnp.dot(h, ow, preferred_element_type=f32) + ob
        outs.append(out)

    out_ref[...] = jnp.concatenate(outs, axis=0)


def kernel(features, conv1_W, conv1_b, conv2_W, conv2_b, conv3_W, conv3_b,
           dense_W, dense_b, out_W, out_b):
    nb = features.shape[0]
    return pl.pallas_call(
        _fwd_kernel,
        out_shape=jax.ShapeDtypeStruct((nb, 2), jnp.float32),
    )(features,
      conv1_W, conv1_b.reshape(1, -1),
      conv2_W, conv2_b.reshape(1, -1),
      conv3_W, conv3_b.reshape(1, -1),
      dense_W, dense_b.reshape(1, -1),
      out_W, out_b.reshape(1, -1))
